# Initial kernel scaffold; baseline (speedup 1.0000x reference)
import functools

import jax
import jax.numpy as jnp
from jax import lax
from jax.experimental import pallas as pl
from jax.experimental.pallas import tpu as pltpu
from jax.experimental.pallas import tpu_sc as plsc

NC = 2
NS = 16
L = 16
W = NC * NS
SUB = 64

CONSTRAINT_WEIGHT = 0.1
CORRECTION_FACTOR = 0.1


@functools.lru_cache(maxsize=None)
def _sc_edge_kernel(E: int, N: int, C: int = 1600):
    assert E % (W * C) == 0 and C % SUB == 0 and C % L == 0 and N % NS == 0
    EPW = E // W
    NCH = EPW // C
    NSUB = C // SUB
    G = C // L
    NPT = N // NS

    mesh = plsc.VectorSubcoreMesh(core_axis_name="c", subcore_axis_name="s")

    @functools.partial(
        pl.kernel,
        out_type=(
            jax.ShapeDtypeStruct((NC, N, 3), jnp.float32),
            jax.ShapeDtypeStruct((NC, NS, L), jnp.float32),
        ),
        mesh=mesh,
        scratch_types=(
            pltpu.VMEM((NSUB, SUB), jnp.int32),
            pltpu.VMEM((NSUB, SUB), jnp.int32),
            pltpu.VMEM((C,), jnp.int32),
            pltpu.VMEM((C, 3), jnp.float32),
            pltpu.VMEM((C, 3), jnp.float32),
            pltpu.VMEM((C, 3), jnp.float32),
            pltpu.VMEM((C, 3), jnp.float32),
            pltpu.VMEM((L,), jnp.float32),
            pltpu.VMEM((L,), jnp.float32),
            pltpu.VMEM((L,), jnp.float32),
            pltpu.VMEM((L,), jnp.float32),
            pltpu.VMEM_SHARED((N, 3), jnp.float32),
            pltpu.SemaphoreType.DMA,
        ),
    )
    def k(row_hbm, col_hbm, bt_hbm, pos_hbm, tgt_hbm, adj_hbm, zeros_hbm,
          acc_out, loss_out,
          rowidx, colidx, btbuf, rbuf, cbuf, adjbuf, nadjbuf,
          tgtv, adjv, tltab, lsbuf, acc, gsem):
        cid = lax.axis_index("c")
        sid = lax.axis_index("s")
        wid = sid * NC + cid

        pltpu.sync_copy(tgt_hbm, tgtv)
        pltpu.sync_copy(adj_hbm, adjv)
        tltab[...] = tgtv[...] + adjv[...]

        pltpu.sync_copy(zeros_hbm, acc.at[pl.ds(sid * NPT, NPT)])
        plsc.subcore_barrier()

        iota = lax.broadcasted_iota(jnp.int32, (L,), 0)
        czero = jnp.zeros((L,), jnp.int32)
        cone = czero + 1
        ctwo = czero + 2

        def chunk_body(kk, loss):
            base = wid * EPW + kk * C
            pltpu.sync_copy(row_hbm.at[pl.ds(base // SUB, NSUB)], rowidx)
            pltpu.sync_copy(col_hbm.at[pl.ds(base // SUB, NSUB)], colidx)
            pltpu.sync_copy(bt_hbm.at[pl.ds(base, C)], btbuf)

            copies = []
            for j in range(NSUB):
                copies.append(pltpu.async_copy(
                    pos_hbm.at[rowidx.at[j]],
                    rbuf.at[pl.ds(j * SUB, SUB)], gsem))
                copies.append(pltpu.async_copy(
                    pos_hbm.at[colidx.at[j]],
                    cbuf.at[pl.ds(j * SUB, SUB)], gsem))
            for h in copies:
                h.wait()

            def g_body(g, ls):
                ridx = iota + g * L
                rx = plsc.load_gather(rbuf, [ridx, czero])
                ry = plsc.load_gather(rbuf, [ridx, cone])
                rz = plsc.load_gather(rbuf, [ridx, ctwo])
                cx = plsc.load_gather(cbuf, [ridx, czero])
                cy = plsc.load_gather(cbuf, [ridx, cone])
                cz = plsc.load_gather(cbuf, [ridx, ctwo])
                bvx = rx - cx
                bvy = ry - cy
                bvz = rz - cz
                l2 = jnp.maximum(bvx * bvx + bvy * bvy + bvz * bvz, 1e-12)
                ii = plsc.bitcast(l2, jnp.int32)
                ii = 0x5F3759DF - lax.shift_right_logical(ii, 1)
                y = plsc.bitcast(ii, jnp.float32)
                xh = 0.5 * l2
                y = y * (1.5 - xh * y * y)
                y = y * (1.5 - xh * y * y)
                y = y * (1.5 - xh * y * y)
                ln = l2 * y
                bt16 = btbuf[pl.ds(pl.multiple_of(g * L, L), L)]
                tl = plsc.load_gather(tltab, [bt16])
                ls = ls + jnp.abs(ln - tl)
                ratio = jnp.clip(tl * y, 0.8, 1.2)
                s = (ratio - 1.0) * (0.5 * CORRECTION_FACTOR)
                ax = bvx * s
                ay = bvy * s
                az = bvz * s
                plsc.store_scatter(adjbuf, [ridx, czero], ax)
                plsc.store_scatter(adjbuf, [ridx, cone], ay)
                plsc.store_scatter(adjbuf, [ridx, ctwo], az)
                plsc.store_scatter(nadjbuf, [ridx, czero], -ax)
                plsc.store_scatter(nadjbuf, [ridx, cone], -ay)
                plsc.store_scatter(nadjbuf, [ridx, ctwo], -az)
                return ls

            loss = lax.fori_loop(0, G, g_body, loss)

            for j in range(NSUB):
                pltpu.sync_copy(adjbuf.at[pl.ds(j * SUB, SUB)],
                                acc.at[rowidx.at[j]], add=True)
                pltpu.sync_copy(nadjbuf.at[pl.ds(j * SUB, SUB)],
                                acc.at[colidx.at[j]], add=True)
            return loss

        loss = lax.fori_loop(0, NCH, chunk_body, jnp.zeros((L,), jnp.float32))

        plsc.subcore_barrier()
        pltpu.sync_copy(acc.at[pl.ds(sid * NPT, NPT)],
                        acc_out.at[cid, pl.ds(sid * NPT, NPT)])
        lsbuf[...] = loss
        pltpu.sync_copy(lsbuf, loss_out.at[cid, sid])

    return k


def kernel(pos, edge_index, bond_types, target_lengths, length_adjustment):
    N = pos.shape[0]
    E = edge_index.shape[1]
    row = edge_index[0].astype(jnp.int32).reshape(E // SUB, SUB)
    col = edge_index[1].astype(jnp.int32).reshape(E // SUB, SUB)
    bt = bond_types.astype(jnp.int32)
    pos32 = pos.astype(jnp.float32)
    tgt16 = jnp.zeros((L,), jnp.float32).at[:5].set(
        target_lengths.astype(jnp.float32))
    adj16 = jnp.zeros((L,), jnp.float32).at[:5].set(
        length_adjustment.astype(jnp.float32))
    zeros = jnp.zeros((N // NS, 3), jnp.float32)

    acc, lpart = _sc_edge_kernel(E, N)(
        row, col, bt, pos32, tgt16, adj16, zeros)

    pos_new = pos32 + acc[0] + acc[1]
    loss = jnp.sum(lpart) * (CONSTRAINT_WEIGHT / E)
    return pos_new, loss


# speedup vs baseline: 59.6478x; 59.6478x over previous
import functools

import jax
import jax.numpy as jnp
from jax import lax
from jax.experimental import pallas as pl
from jax.experimental.pallas import tpu as pltpu
from jax.experimental.pallas import tpu_sc as plsc

NC = 2
NS = 16
L = 16
W = NC * NS
SUB = 64

CONSTRAINT_WEIGHT = 0.1
CORRECTION_FACTOR = 0.1


@functools.lru_cache(maxsize=None)
def _sc_edge_kernel(E: int, NP: int, C: int = 1600):
    assert E % (W * C) == 0 and C % SUB == 0 and C % L == 0
    assert NP % (NS * 8) == 0
    EPW = E // W
    NCH = EPW // C
    NSUB = C // SUB
    G = C // L
    NPT = NP // NS

    mesh = plsc.VectorSubcoreMesh(
        core_axis_name="c", subcore_axis_name="s",
        num_cores=NC, num_subcores=NS)

    @functools.partial(
        pl.kernel,
        out_type=(
            jax.ShapeDtypeStruct((NC * 3 * NP,), jnp.float32),
            jax.ShapeDtypeStruct((W * L,), jnp.float32),
        ),
        mesh=mesh,
        compiler_params=pltpu.CompilerParams(
            needs_layout_passes=False, use_tc_tiling_on_sc=False),
        scratch_types=(
            pltpu.VMEM((C,), jnp.int32),
            pltpu.VMEM((C,), jnp.int32),
            pltpu.VMEM((C,), jnp.int32),
            pltpu.VMEM((C,), jnp.float32),
            pltpu.VMEM((C,), jnp.float32),
            pltpu.VMEM((C,), jnp.float32),
            pltpu.VMEM((C,), jnp.float32),
            pltpu.VMEM((C,), jnp.float32),
            pltpu.VMEM((C,), jnp.float32),
            pltpu.VMEM((C,), jnp.float32),
            pltpu.VMEM((C,), jnp.float32),
            pltpu.VMEM((C,), jnp.float32),
            pltpu.VMEM((C,), jnp.float32),
            pltpu.VMEM((C,), jnp.float32),
            pltpu.VMEM((C,), jnp.float32),
            pltpu.VMEM((L,), jnp.float32),
            pltpu.VMEM((L,), jnp.float32),
            pltpu.VMEM((L,), jnp.float32),
            pltpu.VMEM((L,), jnp.float32),
            pltpu.VMEM_SHARED((NP,), jnp.float32),
            pltpu.VMEM_SHARED((NP,), jnp.float32),
            pltpu.VMEM_SHARED((NP,), jnp.float32),
            pltpu.VMEM_SHARED((NP,), jnp.float32),
            pltpu.VMEM_SHARED((NP,), jnp.float32),
            pltpu.VMEM_SHARED((NP,), jnp.float32),
            pltpu.SemaphoreType.DMA,
            pltpu.SemaphoreType.DMA,
        ),
    )
    def k(px_hbm, py_hbm, pz_hbm, row_hbm, col_hbm, bt_hbm,
          tgt_hbm, adj_hbm, zeros_hbm,
          acc_out, loss_out,
          rowidx, colidx, btbuf,
          rxb, ryb, rzb, cxb, cyb, czb,
          axb, ayb, azb, naxb, nayb, nazb,
          tgtv, adjv, tltab, lsbuf,
          posx, posy, posz, accx, accy, accz,
          gsem, ssem):
        cid = lax.axis_index("c")
        sid = lax.axis_index("s")
        wid = sid * NC + cid

        pltpu.sync_copy(tgt_hbm, tgtv)
        pltpu.sync_copy(adj_hbm, adjv)
        tltab[...] = tgtv[...] + adjv[...]

        slab = pl.ds(sid * NPT, NPT)
        pltpu.sync_copy(px_hbm.at[slab], posx.at[slab])
        pltpu.sync_copy(py_hbm.at[slab], posy.at[slab])
        pltpu.sync_copy(pz_hbm.at[slab], posz.at[slab])
        pltpu.sync_copy(zeros_hbm, accx.at[slab])
        pltpu.sync_copy(zeros_hbm, accy.at[slab])
        pltpu.sync_copy(zeros_hbm, accz.at[slab])
        plsc.subcore_barrier()

        def chunk_body(kk, loss):
            base = wid * EPW + kk * C
            pltpu.sync_copy(row_hbm.at[pl.ds(base, C)], rowidx)
            pltpu.sync_copy(col_hbm.at[pl.ds(base, C)], colidx)
            pltpu.sync_copy(bt_hbm.at[pl.ds(base, C)], btbuf)

            copies = []
            for j in range(NSUB):
                d = pl.ds(j * SUB, SUB)
                ri = rowidx.at[d]
                ci = colidx.at[d]
                for src, dst in ((posx, rxb), (posy, ryb), (posz, rzb)):
                    copies.append(pltpu.async_copy(src.at[ri], dst.at[d], gsem))
                for src, dst in ((posx, cxb), (posy, cyb), (posz, czb)):
                    copies.append(pltpu.async_copy(src.at[ci], dst.at[d], gsem))
            for h in copies:
                h.wait()

            def g_body(g, ls):
                o = pl.ds(pl.multiple_of(g * L, L), L)
                bvx = rxb[o] - cxb[o]
                bvy = ryb[o] - cyb[o]
                bvz = rzb[o] - czb[o]
                l2 = jnp.maximum(bvx * bvx + bvy * bvy + bvz * bvz, 1e-12)
                ii = plsc.bitcast(l2, jnp.int32)
                ii = 0x5F3759DF - lax.shift_right_logical(ii, 1)
                y = plsc.bitcast(ii, jnp.float32)
                xh = 0.5 * l2
                y = y * (1.5 - xh * y * y)
                y = y * (1.5 - xh * y * y)
                y = y * (1.5 - xh * y * y)
                ln = l2 * y
                tl = plsc.load_gather(tltab, [btbuf[o]])
                ls = ls + jnp.abs(ln - tl)
                ratio = jnp.clip(tl * y, 0.8, 1.2)
                s = (ratio - 1.0) * (0.5 * CORRECTION_FACTOR)
                ax = bvx * s
                ay = bvy * s
                az = bvz * s
                axb[o] = ax
                ayb[o] = ay
                azb[o] = az
                naxb[o] = -ax
                nayb[o] = -ay
                nazb[o] = -az
                return ls

            loss = lax.fori_loop(0, G, g_body, loss)

            scatters = []
            for j in range(NSUB):
                d = pl.ds(j * SUB, SUB)
                ri = rowidx.at[d]
                ci = colidx.at[d]
                for src, dst in ((axb, accx), (ayb, accy), (azb, accz)):
                    scatters.append(pltpu.async_copy(
                        src.at[d], dst.at[ri], ssem, add=True))
                for src, dst in ((naxb, accx), (nayb, accy), (nazb, accz)):
                    scatters.append(pltpu.async_copy(
                        src.at[d], dst.at[ci], ssem, add=True))
            for h in scatters:
                h.wait()
            return loss

        loss = lax.fori_loop(0, NCH, chunk_body, jnp.zeros((L,), jnp.float32))

        plsc.subcore_barrier()
        obase = cid * 3 * NP + sid * NPT
        pltpu.sync_copy(accx.at[slab], acc_out.at[pl.ds(obase, NPT)])
        pltpu.sync_copy(accy.at[slab], acc_out.at[pl.ds(obase + NP, NPT)])
        pltpu.sync_copy(accz.at[slab], acc_out.at[pl.ds(obase + 2 * NP, NPT)])
        lsbuf[...] = loss
        pltpu.sync_copy(lsbuf, loss_out.at[pl.ds((cid * NS + sid) * L, L)])

    return k


def kernel(pos, edge_index, bond_types, target_lengths, length_adjustment):
    N = pos.shape[0]
    E = edge_index.shape[1]
    NP = -(-N // (NS * 8)) * NS * 8
    pos32 = pos.astype(jnp.float32)
    planes = jnp.pad(pos32.T, ((0, 0), (0, NP - N)))
    row = edge_index[0].astype(jnp.int32)
    col = edge_index[1].astype(jnp.int32)
    bt = bond_types.astype(jnp.int32)
    tgt16 = jnp.zeros((L,), jnp.float32).at[:5].set(
        target_lengths.astype(jnp.float32))
    adj16 = jnp.zeros((L,), jnp.float32).at[:5].set(
        length_adjustment.astype(jnp.float32))
    zeros = jnp.zeros((NP // NS,), jnp.float32)

    acc, lpart = _sc_edge_kernel(E, NP)(
        planes[0], planes[1], planes[2], row, col, bt, tgt16, adj16, zeros)

    acc = acc.reshape(NC, 3, NP)
    pos_new = pos32 + (acc[0, :, :N] + acc[1, :, :N]).T
    loss = jnp.sum(lpart) * (CONSTRAINT_WEIGHT / E)
    return pos_new, loss


# speedup vs baseline: 61.9166x; 1.0380x over previous
import functools

import jax
import jax.numpy as jnp
from jax import lax
from jax.experimental import pallas as pl
from jax.experimental.pallas import tpu as pltpu
from jax.experimental.pallas import tpu_sc as plsc

NC = 2
NS = 16
L = 16
W = NC * NS
SUB = 64

CONSTRAINT_WEIGHT = 0.1
CORRECTION_FACTOR = 0.1


@functools.lru_cache(maxsize=None)
def _sc_edge_kernel(E: int, NP: int, C: int = 1600):
    assert E % (W * C) == 0 and C % SUB == 0 and C % L == 0
    assert NP % (NS * 8) == 0
    EPW = E // W
    NCH = EPW // C
    NSUB = C // SUB
    G = C // L
    NPT = NP // NS

    mesh = plsc.VectorSubcoreMesh(
        core_axis_name="c", subcore_axis_name="s",
        num_cores=NC, num_subcores=NS)

    @functools.partial(
        pl.kernel,
        out_type=(
            jax.ShapeDtypeStruct((NC * 3 * NP,), jnp.float32),
            jax.ShapeDtypeStruct((W * L,), jnp.float32),
        ),
        mesh=mesh,
        compiler_params=pltpu.CompilerParams(
            needs_layout_passes=False, use_tc_tiling_on_sc=False),
        scratch_types=(
            pltpu.VMEM((C,), jnp.int32),
            pltpu.VMEM((C,), jnp.int32),
            pltpu.VMEM((C,), jnp.int32),
            pltpu.VMEM((C,), jnp.float32),
            pltpu.VMEM((C,), jnp.float32),
            pltpu.VMEM((C,), jnp.float32),
            pltpu.VMEM((C,), jnp.float32),
            pltpu.VMEM((C,), jnp.float32),
            pltpu.VMEM((C,), jnp.float32),
            pltpu.VMEM((C,), jnp.float32),
            pltpu.VMEM((C,), jnp.float32),
            pltpu.VMEM((C,), jnp.float32),
            pltpu.VMEM((C,), jnp.float32),
            pltpu.VMEM((C,), jnp.float32),
            pltpu.VMEM((C,), jnp.float32),
            pltpu.VMEM((L,), jnp.float32),
            pltpu.VMEM((L,), jnp.float32),
            pltpu.VMEM((L,), jnp.float32),
            pltpu.VMEM((L,), jnp.float32),
            pltpu.VMEM_SHARED((NP,), jnp.float32),
            pltpu.VMEM_SHARED((NP,), jnp.float32),
            pltpu.VMEM_SHARED((NP,), jnp.float32),
            pltpu.VMEM_SHARED((NP,), jnp.float32),
            pltpu.VMEM_SHARED((NP,), jnp.float32),
            pltpu.VMEM_SHARED((NP,), jnp.float32),
            pltpu.SemaphoreType.DMA,
            pltpu.SemaphoreType.DMA,
        ),
    )
    def k(px_hbm, py_hbm, pz_hbm, row_hbm, col_hbm, bt_hbm,
          tgt_hbm, adj_hbm, zeros_hbm,
          acc_out, loss_out,
          rowidx, colidx, btbuf,
          rxb, ryb, rzb, cxb, cyb, czb,
          axb, ayb, azb, naxb, nayb, nazb,
          tgtv, adjv, tltab, lsbuf,
          posx, posy, posz, accx, accy, accz,
          gsem, ssem):
        cid = lax.axis_index("c")
        sid = lax.axis_index("s")
        wid = sid * NC + cid

        pltpu.sync_copy(tgt_hbm, tgtv)
        pltpu.sync_copy(adj_hbm, adjv)
        tltab[...] = tgtv[...] + adjv[...]

        slab = pl.ds(sid * NPT, NPT)
        pltpu.sync_copy(px_hbm.at[slab], posx.at[slab])
        pltpu.sync_copy(py_hbm.at[slab], posy.at[slab])
        pltpu.sync_copy(pz_hbm.at[slab], posz.at[slab])
        pltpu.sync_copy(zeros_hbm, accx.at[slab])
        pltpu.sync_copy(zeros_hbm, accy.at[slab])
        pltpu.sync_copy(zeros_hbm, accz.at[slab])
        plsc.subcore_barrier()

        def chunk_body(kk, loss):
            base = wid * EPW + kk * C
            pltpu.sync_copy(row_hbm.at[pl.ds(base, C)], rowidx)
            pltpu.sync_copy(col_hbm.at[pl.ds(base, C)], colidx)
            pltpu.sync_copy(bt_hbm.at[pl.ds(base, C)], btbuf)

            copies = []
            for src, dst in ((posx, rxb), (posy, ryb), (posz, rzb)):
                copies.append(pltpu.async_copy(src.at[rowidx], dst, gsem))
            for src, dst in ((posx, cxb), (posy, cyb), (posz, czb)):
                copies.append(pltpu.async_copy(src.at[colidx], dst, gsem))
            for h in copies:
                h.wait()

            def g_body(g, ls):
                o = pl.ds(pl.multiple_of(g * L, L), L)
                bvx = rxb[o] - cxb[o]
                bvy = ryb[o] - cyb[o]
                bvz = rzb[o] - czb[o]
                l2 = jnp.maximum(bvx * bvx + bvy * bvy + bvz * bvz, 1e-12)
                ii = plsc.bitcast(l2, jnp.int32)
                ii = 0x5F3759DF - lax.shift_right_logical(ii, 1)
                y = plsc.bitcast(ii, jnp.float32)
                xh = 0.5 * l2
                y = y * (1.5 - xh * y * y)
                y = y * (1.5 - xh * y * y)
                y = y * (1.5 - xh * y * y)
                ln = l2 * y
                tl = plsc.load_gather(tltab, [btbuf[o]])
                ls = ls + jnp.abs(ln - tl)
                ratio = jnp.clip(tl * y, 0.8, 1.2)
                s = (ratio - 1.0) * (0.5 * CORRECTION_FACTOR)
                ax = bvx * s
                ay = bvy * s
                az = bvz * s
                axb[o] = ax
                ayb[o] = ay
                azb[o] = az
                naxb[o] = -ax
                nayb[o] = -ay
                nazb[o] = -az
                return ls

            loss = lax.fori_loop(0, G, g_body, loss)

            scatters = []
            for src, dst in ((axb, accx), (ayb, accy), (azb, accz)):
                scatters.append(pltpu.async_copy(
                    src, dst.at[rowidx], ssem, add=True))
            for src, dst in ((naxb, accx), (nayb, accy), (nazb, accz)):
                scatters.append(pltpu.async_copy(
                    src, dst.at[colidx], ssem, add=True))
            for h in scatters:
                h.wait()
            return loss

        loss = lax.fori_loop(0, NCH, chunk_body, jnp.zeros((L,), jnp.float32))

        plsc.subcore_barrier()
        obase = cid * 3 * NP + sid * NPT
        pltpu.sync_copy(accx.at[slab], acc_out.at[pl.ds(obase, NPT)])
        pltpu.sync_copy(accy.at[slab], acc_out.at[pl.ds(obase + NP, NPT)])
        pltpu.sync_copy(accz.at[slab], acc_out.at[pl.ds(obase + 2 * NP, NPT)])
        lsbuf[...] = loss
        pltpu.sync_copy(lsbuf, loss_out.at[pl.ds((cid * NS + sid) * L, L)])

    return k


def kernel(pos, edge_index, bond_types, target_lengths, length_adjustment):
    N = pos.shape[0]
    E = edge_index.shape[1]
    NP = -(-N // (NS * 8)) * NS * 8
    pos32 = pos.astype(jnp.float32)
    planes = jnp.pad(pos32.T, ((0, 0), (0, NP - N)))
    row = edge_index[0].astype(jnp.int32)
    col = edge_index[1].astype(jnp.int32)
    bt = bond_types.astype(jnp.int32)
    tgt16 = jnp.zeros((L,), jnp.float32).at[:5].set(
        target_lengths.astype(jnp.float32))
    adj16 = jnp.zeros((L,), jnp.float32).at[:5].set(
        length_adjustment.astype(jnp.float32))
    zeros = jnp.zeros((NP // NS,), jnp.float32)

    acc, lpart = _sc_edge_kernel(E, NP)(
        planes[0], planes[1], planes[2], row, col, bt, tgt16, adj16, zeros)

    acc = acc.reshape(NC, 3, NP)
    pos_new = pos32 + (acc[0, :, :N] + acc[1, :, :N]).T
    loss = jnp.sum(lpart) * (CONSTRAINT_WEIGHT / E)
    return pos_new, loss
